# baseline (speedup 1.0000x reference)
import jax
import jax.numpy as jnp
from jax import lax
from jax.experimental import pallas as pl
from jax.experimental.pallas import tpu as pltpu
from jax.experimental.pallas import tpu_sc as plsc

N_ROWS = 16384
N_COLS = 26
D = 64
B = N_ROWS * N_COLS
V = 1000000
VH = V // 2

NC = 2
NS = 16
NW = NC * NS

B_PER_W = B // NW
CH = 128
N_CHUNKS = B_PER_W // CH
assert N_CHUNKS * CH == B_PER_W
assert N_ROWS % CH == 0


def _gather_body(table_hbm, idx_hbm, out_hbm, idx_v, idxh_v, rows, tbuf,
                 gsem0, gsem1, ssem0, ssem1):
    c = lax.axis_index("c")
    s = lax.axis_index("s")
    wid = s * NC + c
    base = wid * B_PER_W

    pltpu.sync_copy(idx_hbm.at[wid], idx_v)

    @plsc.parallel_loop(0, N_CHUNKS, step=1)
    def _(r):
        for kk in range(8):
            idxh_v[r, pl.ds(16 * kk, 16)] = jnp.right_shift(
                idx_v[r, pl.ds(16 * kk, 16)], 1)

    rowsb = (rows.at[0], rows.at[1])
    tbufs = (tbuf.at[0], tbuf.at[1])
    gsems = (gsem0, gsem1)
    ssems = (ssem0, ssem1)
    row16 = [lax.iota(jnp.int32, 16) + 16 * kk for kk in range(8)]

    @pl.loop(0, N_CHUNKS, step=2)
    def _(k):
        gs = []
        for bsel in range(2):
            gs.append(pltpu.async_copy(
                table_hbm.at[idxh_v.at[k + bsel]],
                rowsb[bsel], gsems[bsel]))
        ss = []
        for bsel in range(2):
            p = base + (k + bsel) * CH
            j = p // N_ROWS
            r0 = p % N_ROWS
            par = [
                jnp.left_shift(
                    jnp.bitwise_and(
                        idx_v[k + bsel, pl.ds(16 * kk, 16)], 1),
                    6)
                for kk in range(8)
            ]
            gs[bsel].wait()

            @plsc.parallel_loop(0, D, step=1, unroll=4)
            def _(d):
                for kk in range(8):
                    cols = par[kk] + d
                    vals = plsc.load_gather(rowsb[bsel], [row16[kk], cols])
                    tbufs[bsel][d, pl.ds(16 * kk, 16)] = vals

            ss.append(pltpu.async_copy(
                tbufs[bsel], out_hbm.at[j, :, pl.ds(r0, CH)], ssems[bsel]))
        for cp in ss:
            cp.wait()


@jax.jit
def _gather(xt_grouped, table_pairs):
    mesh = plsc.VectorSubcoreMesh(
        core_axis_name="c", subcore_axis_name="s", num_cores=NC, num_subcores=NS
    )
    run = pl.kernel(
        _gather_body,
        out_type=jax.ShapeDtypeStruct((N_COLS, D, N_ROWS), jnp.float32),
        mesh=mesh,
        scratch_types=[
            pltpu.VMEM((N_CHUNKS, CH), jnp.int32),
            pltpu.VMEM((N_CHUNKS, CH), jnp.int32),
            pltpu.VMEM((2, CH, 2 * D), jnp.float32),
            pltpu.VMEM((2, D, CH), jnp.float32),
            pltpu.SemaphoreType.DMA,
            pltpu.SemaphoreType.DMA,
            pltpu.SemaphoreType.DMA,
            pltpu.SemaphoreType.DMA,
        ],
        compiler_params=pltpu.CompilerParams(
            use_tc_tiling_on_sc=True, needs_layout_passes=False
        ),
    )
    return run(table_pairs, xt_grouped)


def kernel(x, table):
    table_pairs = jnp.reshape(table, (VH, 2 * D))
    xt_grouped = jnp.reshape(jnp.transpose(x).astype(jnp.int32), (NW, N_CHUNKS, CH))
    out_t = _gather(xt_grouped, table_pairs)
    return jnp.transpose(out_t, (2, 0, 1))


# speedup vs baseline: 1.3212x; 1.3212x over previous
import jax
import jax.numpy as jnp
from jax import lax
from jax.experimental import pallas as pl
from jax.experimental.pallas import tpu as pltpu
from jax.experimental.pallas import tpu_sc as plsc

N_ROWS = 16384
N_COLS = 26
D = 64
B = N_ROWS * N_COLS

NC = 2
NS = 16
NW = NC * NS

B_PER_W = B // NW
CH = 128
N_CHUNKS = B_PER_W // CH
assert N_CHUNKS * CH == B_PER_W
assert N_ROWS % CH == 0


def _body(table_hbm, idx_hbm, out_hbm, idx_v, rows, gsem0, gsem1, ssem0, ssem1):
    c = lax.axis_index("c")
    s = lax.axis_index("s")
    wid = s * NC + c
    base = wid * B_PER_W

    pltpu.sync_copy(idx_hbm.at[wid], idx_v)

    @pl.loop(0, N_CHUNKS, step=2)
    def _(k):
        g0 = pltpu.async_copy(table_hbm.at[idx_v.at[k]], rows.at[0], gsem0)
        g1 = pltpu.async_copy(table_hbm.at[idx_v.at[k + 1]], rows.at[1], gsem1)
        p0 = base + k * CH
        j0 = p0 // N_ROWS
        r0 = p0 % N_ROWS
        p1 = p0 + CH
        j1 = p1 // N_ROWS
        r1 = p1 % N_ROWS
        g0.wait()
        s0 = pltpu.async_copy(
            rows.at[0], out_hbm.at[pl.ds(r0, CH), j0, pl.ds(0, D)], ssem0)
        g1.wait()
        s1 = pltpu.async_copy(
            rows.at[1], out_hbm.at[pl.ds(r1, CH), j1, pl.ds(0, D)], ssem1)
        s0.wait()
        s1.wait()


@jax.jit
def _gather(xt_grouped, table):
    mesh = plsc.VectorSubcoreMesh(
        core_axis_name="c", subcore_axis_name="s", num_cores=NC, num_subcores=NS
    )
    run = pl.kernel(
        _body,
        out_type=jax.ShapeDtypeStruct((N_ROWS, 32, 128), jnp.float32),
        mesh=mesh,
        scratch_types=[
            pltpu.VMEM((N_CHUNKS, CH), jnp.int32),
            pltpu.VMEM((2, CH, D), jnp.float32),
            pltpu.SemaphoreType.DMA,
            pltpu.SemaphoreType.DMA,
            pltpu.SemaphoreType.DMA,
            pltpu.SemaphoreType.DMA,
        ],
        compiler_params=pltpu.CompilerParams(use_tc_tiling_on_sc=False),
    )
    return run(table, xt_grouped)


def kernel(x, table):
    xt_grouped = jnp.reshape(jnp.transpose(x).astype(jnp.int32), (NW, N_CHUNKS, CH))
    out_pad = _gather(xt_grouped, table)
    return lax.slice(out_pad, (0, 0, 0), (N_ROWS, N_COLS, D))
